# baseline (speedup 1.0000x reference)
import functools
import jax
import jax.numpy as jnp
from jax.experimental import pallas as pl
from jax.experimental.pallas import tpu as pltpu

_B, _C, _FH, _FW = 2, 256, 64, 64
_A = 9
_PRE_N, _POST_N, _IOU_THR = 2000, 1000, 0.7
_NPAD = 2048
_BLK = 128
_NBLK = _NPAD // _BLK

_INTERPRET = False


def _nms_body(bt_ref, x1c_ref, y1c_ref, x2c_ref, y2c_ref, sc_ref, out_ref):
    x1r = bt_ref[0, 0:1, :]
    y1r = bt_ref[0, 1:2, :]
    x2r = bt_ref[0, 2:3, :]
    y2r = bt_ref[0, 3:4, :]
    arear = (x2r - x1r) * (y2r - y1r)
    gcol = jax.lax.broadcasted_iota(jnp.int32, (1, _NPAD), 1)
    rio = jax.lax.broadcasted_iota(jnp.int32, (_BLK, _BLK), 0)
    cio = jax.lax.broadcasted_iota(jnp.int32, (_BLK, _BLK), 1)
    tri = cio > rio
    dead = jnp.zeros((1, _NPAD), jnp.float32)
    keeps = []
    for b in range(_NBLK):
        base = b * _BLK
        x1c = x1c_ref[0, pl.ds(base, _BLK), :]
        y1c = y1c_ref[0, pl.ds(base, _BLK), :]
        x2c = x2c_ref[0, pl.ds(base, _BLK), :]
        y2c = y2c_ref[0, pl.ds(base, _BLK), :]
        iw = jnp.maximum(jnp.minimum(x2c, x2r) - jnp.maximum(x1c, x1r), 0.0)
        ih = jnp.maximum(jnp.minimum(y2c, y2r) - jnp.maximum(y1c, y1r), 0.0)
        inter = iw * ih
        areac = (x2c - x1c) * (y2c - y1c)
        thr = 0.7 * (areac + arear + 1e-8)
        swide = jnp.where(1.7 * inter > thr, 1.0, 0.0)
        supblk = jnp.where(tri, swide[:, base:base + _BLK], 0.0)
        keep0 = 1.0 - dead[0:1, base:base + _BLK]

        def w_cond(c):
            return c[1]

        def w_body(c):
            x = c[0]
            xb = jnp.broadcast_to(x, (8, _BLK))
            cnt = jax.lax.dot_general(
                xb, supblk, (((1,), (0,)), ((), ())),
                preferred_element_type=jnp.float32)[0:1, :]
            xn = jnp.where((cnt < 0.5) & (keep0 > 0.5), 1.0, 0.0)
            return (xn, jnp.any(xn != x))

        keep, _ = jax.lax.while_loop(w_cond, w_body,
                                     (keep0, jnp.bool_(True)))
        keeps.append(keep)
        if b < _NBLK - 1:
            km = jnp.broadcast_to(keep, (8, _BLK))
            cnt = jax.lax.dot_general(km, swide, (((1,), (0,)), ((), ())),
                                      preferred_element_type=jnp.float32)
            live = (cnt[0:1, :] > 0.5) & (gcol >= base + _BLK)
            dead = jnp.maximum(dead, jnp.where(live, 1.0, 0.0))
    keep_full = jnp.concatenate(keeps, axis=1)
    sc = sc_ref[0]
    out_ref[0] = jnp.where(keep_full > 0.5, sc, -jnp.inf)


def _nms_pallas(bt, x1c, y1c, x2c, y2c, sc):
    spec3 = lambda shape: pl.BlockSpec(shape, lambda i: (i, 0, 0))
    return pl.pallas_call(
        _nms_body,
        grid=(_B,),
        in_specs=[
            spec3((1, 4, _NPAD)),
            spec3((1, _NPAD, 1)),
            spec3((1, _NPAD, 1)),
            spec3((1, _NPAD, 1)),
            spec3((1, _NPAD, 1)),
            spec3((1, 1, _NPAD)),
        ],
        out_specs=spec3((1, 1, _NPAD)),
        out_shape=jax.ShapeDtypeStruct((_B, 1, _NPAD), jnp.float32),
        interpret=_INTERPRET,
    )(bt, x1c, y1c, x2c, y2c, sc)


_HW = _FH * _FW
_RH = 512
_NSTRIP = _HW // _RH
_PADR = 65


def _rpn_front_body(xp_ref, w1_ref, wh_ref, b1_ref, bh_ref, ca_ref, sa_ref,
                    imsz_ref, obj_ref, prop_ref):
    b1r = b1_ref[0:1, :]
    bhr = bh_ref[0:1, :]
    imv = imsz_ref[0, :, :]
    rio = jax.lax.broadcasted_iota(jnp.int32, (_RH, 1), 0)
    w_of_r = jax.lax.rem(rio, _FW)
    m0 = jnp.where(w_of_r == 0, 0.0, 1.0)
    m2 = jnp.where(w_of_r == _FW - 1, 0.0, 1.0)
    jl = jax.lax.broadcasted_iota(jnp.int32, (1, 36), 1) % 4
    sgn = jnp.where(jl < 2, -0.5, 0.5)
    for i in range(_NSTRIP):
        acc = jnp.zeros((_RH, _C), jnp.float32)
        for dy in range(3):
            for dx in range(3):
                start = i * _RH + dy * _FW + dx
                xs = xp_ref[0, start:start + _RH, :]
                if dx == 0:
                    xs = xs * m0
                elif dx == 2:
                    xs = xs * m2
                wt = w1_ref[(dy * 3 + dx) * _C:(dy * 3 + dx + 1) * _C, :]
                acc = acc + jax.lax.dot_general(
                    xs, wt, (((1,), (0,)), ((), ())),
                    preferred_element_type=jnp.float32)
        t = jnp.maximum(acc + b1r, 0.0)
        head = jax.lax.dot_general(t, wh_ref[:, :],
                                   (((1,), (0,)), ((), ())),
                                   preferred_element_type=jnp.float32) + bhr
        logits = head[:, 36:45]
        z = jnp.exp(-jnp.abs(logits))
        sig = jnp.where(logits >= 0, 1.0 / (1.0 + z), z / (1.0 + z))
        obj_ref[0, i * _RH:(i + 1) * _RH, :] = sig
        a = head[:, 0:36]
        a2 = jnp.concatenate([a[:, 2:], a[:, :2]], axis=1)
        am2 = jnp.concatenate([a[:, 34:], a[:, :34]], axis=1)
        b2 = jnp.where(jl < 2, a2, a)
        b0 = jnp.where(jl < 2, a, am2)
        ca = ca_ref[0, i * _RH:(i + 1) * _RH, :]
        sa = sa_ref[0, i * _RH:(i + 1) * _RH, :]
        e = jnp.exp(jnp.minimum(b2, 4.135)) * sa
        prop = b0 * sa + ca + sgn * e
        prop_ref[0, i * _RH:(i + 1) * _RH, :] = jnp.minimum(
            jnp.maximum(prop, 0.0), imv)


def _rpn_front(xp, w1t, wh, b1, bh, ca, sa, imsz):
    bspec = lambda shape: pl.BlockSpec(shape, lambda i: (i,) + (0,) * (len(shape) - 1))
    cspec = lambda shape: pl.BlockSpec(shape, lambda i: (0,) * len(shape))
    return pl.pallas_call(
        _rpn_front_body,
        grid=(_B,),
        in_specs=[
            bspec((1, _HW + 2 * _PADR, _C)),
            cspec((9 * _C, _C)),
            cspec((_C, 45)),
            cspec((1, _C)),
            cspec((1, 45)),
            cspec((1, _HW, 36)),
            cspec((1, _HW, 36)),
            cspec((1, 1, 1)),
        ],
        out_specs=[bspec((1, _HW, 9)), bspec((1, _HW, 36))],
        out_shape=[
            jax.ShapeDtypeStruct((_B, _HW, 9), jnp.float32),
            jax.ShapeDtypeStruct((_B, _HW, 36), jnp.float32),
        ],
        interpret=_INTERPRET,
    )(xp, w1t, wh, b1, bh, ca, sa, imsz)


def _conv_xla(x, w, b):
    out = jax.lax.conv_general_dilated(
        x, w, (1, 1), 'SAME', dimension_numbers=('NCHW', 'OIHW', 'NCHW'))
    return out + b[None, :, None, None]


def _mk_anchors(image_size, fh, fw):
    sizes = jnp.array([32.0, 64.0, 128.0], dtype=jnp.float32)
    ratios = jnp.array([0.5, 1.0, 2.0], dtype=jnp.float32)
    ws = (sizes[:, None] * jnp.sqrt(ratios)[None, :]).reshape(-1)
    hs = (sizes[:, None] / jnp.sqrt(ratios)[None, :]).reshape(-1)
    sy = image_size / fh
    sx = image_size / fw
    cy = (jnp.arange(fh, dtype=jnp.float32) + 0.5) * sy
    cx = (jnp.arange(fw, dtype=jnp.float32) + 0.5) * sx
    cyg, cxg = jnp.meshgrid(cy, cx, indexing='ij')
    x1 = cxg[:, :, None] - ws[None, None, :] * 0.5
    y1 = cyg[:, :, None] - hs[None, None, :] * 0.5
    x2 = cxg[:, :, None] + ws[None, None, :] * 0.5
    y2 = cyg[:, :, None] + hs[None, None, :] * 0.5
    return jnp.stack([x1, y1, x2, y2], axis=-1).reshape(-1, 4)


def _decode(anchors, deltas):
    wa = anchors[:, 2] - anchors[:, 0]
    ha = anchors[:, 3] - anchors[:, 1]
    cxa = anchors[:, 0] + 0.5 * wa
    cya = anchors[:, 1] + 0.5 * ha
    dx, dy = deltas[:, 0], deltas[:, 1]
    dw = jnp.minimum(deltas[:, 2], 4.135)
    dh = jnp.minimum(deltas[:, 3], 4.135)
    cx = dx * wa + cxa
    cy = dy * ha + cya
    w = jnp.exp(dw) * wa
    h = jnp.exp(dh) * ha
    return jnp.stack(
        [cx - 0.5 * w, cy - 0.5 * h, cx + 0.5 * w, cy + 0.5 * h], axis=1)


def kernel(features, w1, b1, w_cls, b_cls, w_reg, b_reg, image_size):
    bsz = features.shape[0]
    image_size_f = jnp.asarray(image_size, dtype=jnp.float32)

    x = jnp.transpose(features, (0, 2, 3, 1)).reshape(bsz, _HW, _C)
    xp = jnp.pad(x, ((0, 0), (_PADR, _PADR), (0, 0)))
    w1t = jnp.transpose(w1, (2, 3, 1, 0)).reshape(9 * _C, _C)
    wh = jnp.concatenate([w_reg[:, :, 0, 0].T, w_cls[:, :, 0, 0].T], axis=1)
    bh = jnp.concatenate([b_reg, b_cls])[None, :]

    anch = _mk_anchors(image_size_f, _FH, _FW)
    wa = (anch[:, 2] - anch[:, 0]).reshape(_HW, _A)
    ha = (anch[:, 3] - anch[:, 1]).reshape(_HW, _A)
    cxa = anch[:, 0].reshape(_HW, _A) + 0.5 * wa
    cya = anch[:, 1].reshape(_HW, _A) + 0.5 * ha
    sa = jnp.stack([wa, ha, wa, ha], axis=-1).reshape(1, _HW, 36)
    ca = jnp.stack([cxa, cya, cxa, cya], axis=-1).reshape(1, _HW, 36)
    imsz = image_size_f.reshape(1, 1, 1)

    obj9, prop36 = _rpn_front(xp, w1t, wh, b1[None, :], bh, ca, sa, imsz)
    obj = obj9.reshape(bsz, _HW * _A)
    proposals = prop36.reshape(bsz, _HW * _A, 4)

    sc, idx = jax.lax.top_k(obj, _PRE_N)
    bsel = jnp.take_along_axis(proposals, idx[..., None], axis=1)
    pad = _NPAD - _PRE_N
    bpad = jnp.pad(bsel, ((0, 0), (0, pad), (0, 0)))
    scpad = jnp.pad(sc, ((0, 0), (0, pad)), constant_values=0.0)
    bt = jnp.transpose(bpad, (0, 2, 1))
    x1c = bpad[:, :, 0:1]
    y1c = bpad[:, :, 1:2]
    x2c = bpad[:, :, 2:3]
    y2c = bpad[:, :, 3:4]
    masked = _nms_pallas(bt, x1c, y1c, x2c, y2c, scpad[:, None, :])
    masked = masked[:, 0, :_PRE_N]
    _, kidx = jax.lax.top_k(masked, _POST_N)
    return jnp.take_along_axis(bsel, kidx[..., None], axis=1)


# speedup vs baseline: 1.1761x; 1.1761x over previous
import functools
import jax
import jax.numpy as jnp
from jax.experimental import pallas as pl
from jax.experimental.pallas import tpu as pltpu

_B, _C, _FH, _FW = 2, 256, 64, 64
_A = 9
_PRE_N, _POST_N, _IOU_THR = 2000, 1000, 0.7
_NPAD = 2048
_BLK = 128
_NBLK = _NPAD // _BLK

_INTERPRET = False


def _nms_body(bt_ref, x1c_ref, y1c_ref, x2c_ref, y2c_ref, sc_ref, out_ref):
    x1r = bt_ref[0, 0:1, :]
    y1r = bt_ref[0, 1:2, :]
    x2r = bt_ref[0, 2:3, :]
    y2r = bt_ref[0, 3:4, :]
    arear = (x2r - x1r) * (y2r - y1r)
    gcol = jax.lax.broadcasted_iota(jnp.int32, (1, _NPAD), 1)
    rio = jax.lax.broadcasted_iota(jnp.int32, (_BLK, _BLK), 0)
    cio = jax.lax.broadcasted_iota(jnp.int32, (_BLK, _BLK), 1)
    tri = cio > rio
    dead = jnp.zeros((1, _NPAD), jnp.float32)
    keeps = []
    for b in range(_NBLK):
        base = b * _BLK
        x1c = x1c_ref[0, pl.ds(base, _BLK), :]
        y1c = y1c_ref[0, pl.ds(base, _BLK), :]
        x2c = x2c_ref[0, pl.ds(base, _BLK), :]
        y2c = y2c_ref[0, pl.ds(base, _BLK), :]
        iw = jnp.maximum(jnp.minimum(x2c, x2r) - jnp.maximum(x1c, x1r), 0.0)
        ih = jnp.maximum(jnp.minimum(y2c, y2r) - jnp.maximum(y1c, y1r), 0.0)
        inter = iw * ih
        areac = (x2c - x1c) * (y2c - y1c)
        thr = 0.7 * (areac + arear + 1e-8)
        swide = jnp.where(1.7 * inter > thr, 1.0, 0.0)
        supblk = jnp.where(tri, swide[:, base:base + _BLK], 0.0)
        keep0 = 1.0 - dead[0:1, base:base + _BLK]

        def w_cond(c):
            return c[1]

        def w_body(c):
            x = c[0]
            xb = jnp.broadcast_to(x, (8, _BLK))
            cnt = jax.lax.dot_general(
                xb, supblk, (((1,), (0,)), ((), ())),
                preferred_element_type=jnp.float32)[0:1, :]
            xn = jnp.where((cnt < 0.5) & (keep0 > 0.5), 1.0, 0.0)
            return (xn, jnp.any(xn != x))

        keep, _ = jax.lax.while_loop(w_cond, w_body,
                                     (keep0, jnp.bool_(True)))
        keeps.append(keep)
        if b < _NBLK - 1:
            km = jnp.broadcast_to(keep, (8, _BLK))
            cnt = jax.lax.dot_general(km, swide, (((1,), (0,)), ((), ())),
                                      preferred_element_type=jnp.float32)
            live = (cnt[0:1, :] > 0.5) & (gcol >= base + _BLK)
            dead = jnp.maximum(dead, jnp.where(live, 1.0, 0.0))
    keep_full = jnp.concatenate(keeps, axis=1)
    sc = sc_ref[0]
    out_ref[0] = jnp.where(keep_full > 0.5, sc, -jnp.inf)


def _nms_pallas(bt, x1c, y1c, x2c, y2c, sc):
    spec3 = lambda shape: pl.BlockSpec(shape, lambda i: (i, 0, 0))
    return pl.pallas_call(
        _nms_body,
        grid=(_B,),
        in_specs=[
            spec3((1, 4, _NPAD)),
            spec3((1, _NPAD, 1)),
            spec3((1, _NPAD, 1)),
            spec3((1, _NPAD, 1)),
            spec3((1, _NPAD, 1)),
            spec3((1, 1, _NPAD)),
        ],
        out_specs=spec3((1, 1, _NPAD)),
        out_shape=jax.ShapeDtypeStruct((_B, 1, _NPAD), jnp.float32),
        interpret=_INTERPRET,
    )(bt, x1c, y1c, x2c, y2c, sc)


_HW = _FH * _FW
_RH = 512
_NSTRIP = _HW // _RH
_PADR = 65


def _rpn_front_body(xp_ref, w1_ref, wh_ref, b1_ref, bh_ref, ca_ref, sa_ref,
                    imsz_ref, obj_ref, prop_ref):
    b1r = b1_ref[0:1, :]
    bhr = bh_ref[0:1, :]
    imv = imsz_ref[0, :, :]
    rio = jax.lax.broadcasted_iota(jnp.int32, (_RH, 1), 0)
    w_of_r = jax.lax.rem(rio, _FW)
    m0 = jnp.where(w_of_r == 0, 0.0, 1.0)
    m2 = jnp.where(w_of_r == _FW - 1, 0.0, 1.0)
    jl = jax.lax.broadcasted_iota(jnp.int32, (1, 36), 1) % 4
    sgn = jnp.where(jl < 2, -0.5, 0.5)
    for i in range(_NSTRIP):
        acc = jnp.zeros((_RH, _C), jnp.float32)
        for dy in range(3):
            for dx in range(3):
                start = i * _RH + dy * _FW + dx
                xs = xp_ref[0, start:start + _RH, :]
                if dx == 0:
                    xs = xs * m0
                elif dx == 2:
                    xs = xs * m2
                wt = w1_ref[(dy * 3 + dx) * _C:(dy * 3 + dx + 1) * _C, :]
                acc = acc + jax.lax.dot_general(
                    xs, wt, (((1,), (0,)), ((), ())),
                    preferred_element_type=jnp.float32)
        t = jnp.maximum(acc + b1r, 0.0)
        head = jax.lax.dot_general(t, wh_ref[:, :],
                                   (((1,), (0,)), ((), ())),
                                   preferred_element_type=jnp.float32) + bhr
        logits = head[:, 36:45]
        z = jnp.exp(-jnp.abs(logits))
        sig = jnp.where(logits >= 0, 1.0 / (1.0 + z), z / (1.0 + z))
        obj_ref[0, i * _RH:(i + 1) * _RH, :] = sig
        a = head[:, 0:36]
        a2 = jnp.concatenate([a[:, 2:], a[:, :2]], axis=1)
        am2 = jnp.concatenate([a[:, 34:], a[:, :34]], axis=1)
        b2 = jnp.where(jl < 2, a2, a)
        b0 = jnp.where(jl < 2, a, am2)
        ca = ca_ref[0, i * _RH:(i + 1) * _RH, :]
        sa = sa_ref[0, i * _RH:(i + 1) * _RH, :]
        e = jnp.exp(jnp.minimum(b2, 4.135)) * sa
        prop = b0 * sa + ca + sgn * e
        prop_ref[0, i * _RH:(i + 1) * _RH, :] = jnp.minimum(
            jnp.maximum(prop, 0.0), imv)


def _rpn_front(xp, w1t, wh, b1, bh, ca, sa, imsz):
    bspec = lambda shape: pl.BlockSpec(shape, lambda i: (i,) + (0,) * (len(shape) - 1))
    cspec = lambda shape: pl.BlockSpec(shape, lambda i: (0,) * len(shape))
    return pl.pallas_call(
        _rpn_front_body,
        grid=(_B,),
        in_specs=[
            bspec((1, _HW + 2 * _PADR, _C)),
            cspec((9 * _C, _C)),
            cspec((_C, 45)),
            cspec((1, _C)),
            cspec((1, 45)),
            cspec((1, _HW, 36)),
            cspec((1, _HW, 36)),
            cspec((1, 1, 1)),
        ],
        out_specs=[bspec((1, _HW, 9)), bspec((1, _HW, 36))],
        out_shape=[
            jax.ShapeDtypeStruct((_B, _HW, 9), jnp.float32),
            jax.ShapeDtypeStruct((_B, _HW, 36), jnp.float32),
        ],
        interpret=_INTERPRET,
    )(xp, w1t, wh, b1, bh, ca, sa, imsz)


def _thr_body(xb_ref, out_ref):
    xbits = xb_ref[0]

    def body(_, c):
        lo, hi = c
        mid = jax.lax.div(lo + hi, jnp.int32(2))
        cnt = jnp.sum(jnp.where(xbits > mid, 1, 0))
        small = cnt < _PRE_N
        return (jnp.where(small, lo, mid), jnp.where(small, mid, hi))

    lo0 = jnp.int32(-1)
    hi0 = jnp.int32(0x3F800000)
    _, hi = jax.lax.fori_loop(0, 31, body, (lo0, hi0))
    n_gt = jnp.sum(jnp.where(xbits > hi, 1, 0))
    n_ge = jnp.sum(jnp.where(xbits >= hi, 1, 0))
    l = jax.lax.broadcasted_iota(jnp.int32, (1, 128), 1)
    out_ref[0] = jnp.where(l == 0, hi, jnp.where(l == 1, n_gt, n_ge))


def _thr_pallas(xbits):
    return pl.pallas_call(
        _thr_body,
        grid=(_B,),
        in_specs=[pl.BlockSpec((1, 288, 128), lambda i: (i, 0, 0))],
        out_specs=pl.BlockSpec((1, 1, 128), lambda i: (i, 0, 0)),
        out_shape=jax.ShapeDtypeStruct((_B, 1, 128), jnp.int32),
        interpret=_INTERPRET,
    )(xbits)


def _topk2000(obj, proposals):
    nf = jnp.float32(-jnp.inf)
    xbits = jax.lax.bitcast_convert_type(obj, jnp.int32).reshape(_B, 288, 128)
    out = _thr_pallas(xbits)
    v = jax.lax.bitcast_convert_type(out[:, 0, 0], jnp.float32)
    n_gt = out[:, 0, 1]
    n_ge = out[:, 0, 2]

    def fast(obj):
        eq = obj == v[:, None]
        gt = obj > v[:, None]
        eqrank = jnp.cumsum(eq.astype(jnp.int32), axis=1)
        win = gt | (eq & (eqrank <= (_PRE_N - n_gt)[:, None]))
        cwin = jnp.cumsum(win.astype(jnp.int32), axis=1)
        p = jnp.broadcast_to(jnp.arange(1, _NPAD + 1, dtype=jnp.int32)[None],
                             (_B, _NPAD))
        pos = jax.vmap(lambda c, q: jnp.searchsorted(c, q))(cwin, p)
        posc = jnp.minimum(pos, _HW * _A - 1).astype(jnp.int32)
        wsc = jnp.where(p <= _PRE_N,
                        jnp.take_along_axis(obj, posc, axis=1), nf)
        sc, order = jax.lax.top_k(wsc, _PRE_N)
        idx = jnp.take_along_axis(posc, order, axis=1)
        return sc, idx

    def slow(obj):
        v, i = jax.lax.top_k(obj, _PRE_N)
        return v, i

    sc, idx = jax.lax.cond(jnp.any(n_ge > _NPAD), slow, fast, obj)
    bsel = jnp.take_along_axis(proposals, idx[..., None], axis=1)
    return sc, bsel


def _conv_xla(x, w, b):
    out = jax.lax.conv_general_dilated(
        x, w, (1, 1), 'SAME', dimension_numbers=('NCHW', 'OIHW', 'NCHW'))
    return out + b[None, :, None, None]


def _mk_anchors(image_size, fh, fw):
    sizes = jnp.array([32.0, 64.0, 128.0], dtype=jnp.float32)
    ratios = jnp.array([0.5, 1.0, 2.0], dtype=jnp.float32)
    ws = (sizes[:, None] * jnp.sqrt(ratios)[None, :]).reshape(-1)
    hs = (sizes[:, None] / jnp.sqrt(ratios)[None, :]).reshape(-1)
    sy = image_size / fh
    sx = image_size / fw
    cy = (jnp.arange(fh, dtype=jnp.float32) + 0.5) * sy
    cx = (jnp.arange(fw, dtype=jnp.float32) + 0.5) * sx
    cyg, cxg = jnp.meshgrid(cy, cx, indexing='ij')
    x1 = cxg[:, :, None] - ws[None, None, :] * 0.5
    y1 = cyg[:, :, None] - hs[None, None, :] * 0.5
    x2 = cxg[:, :, None] + ws[None, None, :] * 0.5
    y2 = cyg[:, :, None] + hs[None, None, :] * 0.5
    return jnp.stack([x1, y1, x2, y2], axis=-1).reshape(-1, 4)


def _decode(anchors, deltas):
    wa = anchors[:, 2] - anchors[:, 0]
    ha = anchors[:, 3] - anchors[:, 1]
    cxa = anchors[:, 0] + 0.5 * wa
    cya = anchors[:, 1] + 0.5 * ha
    dx, dy = deltas[:, 0], deltas[:, 1]
    dw = jnp.minimum(deltas[:, 2], 4.135)
    dh = jnp.minimum(deltas[:, 3], 4.135)
    cx = dx * wa + cxa
    cy = dy * ha + cya
    w = jnp.exp(dw) * wa
    h = jnp.exp(dh) * ha
    return jnp.stack(
        [cx - 0.5 * w, cy - 0.5 * h, cx + 0.5 * w, cy + 0.5 * h], axis=1)


def kernel(features, w1, b1, w_cls, b_cls, w_reg, b_reg, image_size):
    bsz = features.shape[0]
    image_size_f = jnp.asarray(image_size, dtype=jnp.float32)

    x = jnp.transpose(features, (0, 2, 3, 1)).reshape(bsz, _HW, _C)
    xp = jnp.pad(x, ((0, 0), (_PADR, _PADR), (0, 0)))
    w1t = jnp.transpose(w1, (2, 3, 1, 0)).reshape(9 * _C, _C)
    wh = jnp.concatenate([w_reg[:, :, 0, 0].T, w_cls[:, :, 0, 0].T], axis=1)
    bh = jnp.concatenate([b_reg, b_cls])[None, :]

    anch = _mk_anchors(image_size_f, _FH, _FW)
    wa = (anch[:, 2] - anch[:, 0]).reshape(_HW, _A)
    ha = (anch[:, 3] - anch[:, 1]).reshape(_HW, _A)
    cxa = anch[:, 0].reshape(_HW, _A) + 0.5 * wa
    cya = anch[:, 1].reshape(_HW, _A) + 0.5 * ha
    sa = jnp.stack([wa, ha, wa, ha], axis=-1).reshape(1, _HW, 36)
    ca = jnp.stack([cxa, cya, cxa, cya], axis=-1).reshape(1, _HW, 36)
    imsz = image_size_f.reshape(1, 1, 1)

    obj9, prop36 = _rpn_front(xp, w1t, wh, b1[None, :], bh, ca, sa, imsz)
    obj = obj9.reshape(bsz, _HW * _A)
    proposals = prop36.reshape(bsz, _HW * _A, 4)

    sc, bsel = _topk2000(obj, proposals)
    pad = _NPAD - _PRE_N
    bpad = jnp.pad(bsel, ((0, 0), (0, pad), (0, 0)))
    scpad = jnp.pad(sc, ((0, 0), (0, pad)), constant_values=0.0)
    bt = jnp.transpose(bpad, (0, 2, 1))
    x1c = bpad[:, :, 0:1]
    y1c = bpad[:, :, 1:2]
    x2c = bpad[:, :, 2:3]
    y2c = bpad[:, :, 3:4]
    masked = _nms_pallas(bt, x1c, y1c, x2c, y2c, scpad[:, None, :])
    masked = masked[:, 0, :_PRE_N]
    _, kidx = jax.lax.top_k(masked, _POST_N)
    return jnp.take_along_axis(bsel, kidx[..., None], axis=1)


# speedup vs baseline: 1.1817x; 1.0048x over previous
import functools
import jax
import jax.numpy as jnp
from jax.experimental import pallas as pl
from jax.experimental.pallas import tpu as pltpu

_B, _C, _FH, _FW = 2, 256, 64, 64
_A = 9
_PRE_N, _POST_N, _IOU_THR = 2000, 1000, 0.7
_NPAD = 2048
_BLK = 128
_NBLK = _NPAD // _BLK

_INTERPRET = False


def _nms_body(bt_ref, x1c_ref, y1c_ref, x2c_ref, y2c_ref, sc_ref, out_ref):
    x1r = bt_ref[0, 0:1, :]
    y1r = bt_ref[0, 1:2, :]
    x2r = bt_ref[0, 2:3, :]
    y2r = bt_ref[0, 3:4, :]
    arear = (x2r - x1r) * (y2r - y1r)
    gcol = jax.lax.broadcasted_iota(jnp.int32, (1, _NPAD), 1)
    rio = jax.lax.broadcasted_iota(jnp.int32, (_BLK, _BLK), 0)
    cio = jax.lax.broadcasted_iota(jnp.int32, (_BLK, _BLK), 1)
    tri = cio > rio
    dead = jnp.zeros((1, _NPAD), jnp.float32)
    keeps = []
    for b in range(_NBLK):
        base = b * _BLK
        x1c = x1c_ref[0, pl.ds(base, _BLK), :]
        y1c = y1c_ref[0, pl.ds(base, _BLK), :]
        x2c = x2c_ref[0, pl.ds(base, _BLK), :]
        y2c = y2c_ref[0, pl.ds(base, _BLK), :]
        iw = jnp.maximum(jnp.minimum(x2c, x2r) - jnp.maximum(x1c, x1r), 0.0)
        ih = jnp.maximum(jnp.minimum(y2c, y2r) - jnp.maximum(y1c, y1r), 0.0)
        inter = iw * ih
        areac = (x2c - x1c) * (y2c - y1c)
        thr = 0.7 * (areac + arear + 1e-8)
        swide = jnp.where(1.7 * inter > thr, 1.0, 0.0)
        supblk = jnp.where(tri, swide[:, base:base + _BLK], 0.0)
        keep0 = 1.0 - dead[0:1, base:base + _BLK]

        def w_cond(c):
            return c[1]

        def w_body(c):
            x = c[0]
            xb = jnp.broadcast_to(x, (8, _BLK))
            cnt = jax.lax.dot_general(
                xb, supblk, (((1,), (0,)), ((), ())),
                preferred_element_type=jnp.float32)[0:1, :]
            xn = jnp.where((cnt < 0.5) & (keep0 > 0.5), 1.0, 0.0)
            return (xn, jnp.any(xn != x))

        keep, _ = jax.lax.while_loop(w_cond, w_body,
                                     (keep0, jnp.bool_(True)))
        keeps.append(keep)
        if b < _NBLK - 1:
            km = jnp.broadcast_to(keep, (8, _BLK))
            cnt = jax.lax.dot_general(km, swide, (((1,), (0,)), ((), ())),
                                      preferred_element_type=jnp.float32)
            live = (cnt[0:1, :] > 0.5) & (gcol >= base + _BLK)
            dead = jnp.maximum(dead, jnp.where(live, 1.0, 0.0))
    keep_full = jnp.concatenate(keeps, axis=1)
    sc = sc_ref[0]
    out_ref[0] = jnp.where(keep_full > 0.5, sc, -jnp.inf)


def _nms_pallas(bt, x1c, y1c, x2c, y2c, sc):
    spec3 = lambda shape: pl.BlockSpec(shape, lambda i: (i, 0, 0))
    return pl.pallas_call(
        _nms_body,
        grid=(_B,),
        in_specs=[
            spec3((1, 4, _NPAD)),
            spec3((1, _NPAD, 1)),
            spec3((1, _NPAD, 1)),
            spec3((1, _NPAD, 1)),
            spec3((1, _NPAD, 1)),
            spec3((1, 1, _NPAD)),
        ],
        out_specs=spec3((1, 1, _NPAD)),
        out_shape=jax.ShapeDtypeStruct((_B, 1, _NPAD), jnp.float32),
        interpret=_INTERPRET,
    )(bt, x1c, y1c, x2c, y2c, sc)


_HW = _FH * _FW
_RH = 512
_NSTRIP = _HW // _RH
_PADR = 65


def _rpn_front_body(xp_ref, w1_ref, wh_ref, b1_ref, bh_ref, ca_ref, sa_ref,
                    imsz_ref, obj_ref, prop_ref):
    b1r = b1_ref[0:1, :]
    bhr = bh_ref[0:1, :]
    imv = imsz_ref[0, :, :]
    rio = jax.lax.broadcasted_iota(jnp.int32, (_RH, 1), 0)
    w_of_r = jax.lax.rem(rio, _FW)
    m0 = jnp.where(w_of_r == 0, 0.0, 1.0)
    m2 = jnp.where(w_of_r == _FW - 1, 0.0, 1.0)
    jl = jax.lax.broadcasted_iota(jnp.int32, (1, 36), 1) % 4
    sgn = jnp.where(jl < 2, -0.5, 0.5)
    for i in range(_NSTRIP):
        acc = jnp.zeros((_RH, _C), jnp.float32)
        for dy in range(3):
            for dx in range(3):
                start = i * _RH + dy * _FW + dx
                xs = xp_ref[0, start:start + _RH, :]
                if dx == 0:
                    xs = xs * m0
                elif dx == 2:
                    xs = xs * m2
                wt = w1_ref[(dy * 3 + dx) * _C:(dy * 3 + dx + 1) * _C, :]
                acc = acc + jax.lax.dot_general(
                    xs, wt, (((1,), (0,)), ((), ())),
                    preferred_element_type=jnp.float32)
        t = jnp.maximum(acc + b1r, 0.0)
        head = jax.lax.dot_general(t, wh_ref[:, :],
                                   (((1,), (0,)), ((), ())),
                                   preferred_element_type=jnp.float32) + bhr
        logits = head[:, 36:45]
        z = jnp.exp(-jnp.abs(logits))
        sig = jnp.where(logits >= 0, 1.0 / (1.0 + z), z / (1.0 + z))
        obj_ref[0, i * _RH:(i + 1) * _RH, :] = sig
        a = head[:, 0:36]
        a2 = jnp.concatenate([a[:, 2:], a[:, :2]], axis=1)
        am2 = jnp.concatenate([a[:, 34:], a[:, :34]], axis=1)
        b2 = jnp.where(jl < 2, a2, a)
        b0 = jnp.where(jl < 2, a, am2)
        ca = ca_ref[0, i * _RH:(i + 1) * _RH, :]
        sa = sa_ref[0, i * _RH:(i + 1) * _RH, :]
        e = jnp.exp(jnp.minimum(b2, 4.135)) * sa
        prop = b0 * sa + ca + sgn * e
        prop_ref[0, i * _RH:(i + 1) * _RH, :] = jnp.minimum(
            jnp.maximum(prop, 0.0), imv)


def _rpn_front(xp, w1t, wh, b1, bh, ca, sa, imsz):
    bspec = lambda shape: pl.BlockSpec(shape, lambda i: (i,) + (0,) * (len(shape) - 1))
    cspec = lambda shape: pl.BlockSpec(shape, lambda i: (0,) * len(shape))
    return pl.pallas_call(
        _rpn_front_body,
        grid=(_B,),
        in_specs=[
            bspec((1, _HW + 2 * _PADR, _C)),
            cspec((9 * _C, _C)),
            cspec((_C, 45)),
            cspec((1, _C)),
            cspec((1, 45)),
            cspec((1, _HW, 36)),
            cspec((1, _HW, 36)),
            cspec((1, 1, 1)),
        ],
        out_specs=[bspec((1, _HW, 9)), bspec((1, _HW, 36))],
        out_shape=[
            jax.ShapeDtypeStruct((_B, _HW, 9), jnp.float32),
            jax.ShapeDtypeStruct((_B, _HW, 36), jnp.float32),
        ],
        interpret=_INTERPRET,
    )(xp, w1t, wh, b1, bh, ca, sa, imsz)


def _thr_body(xb_ref, out_ref, cwin_ref):
    xbits = xb_ref[0]

    def body(_, c):
        lo, hi = c
        mid = jax.lax.div(lo + hi, jnp.int32(2))
        cnt = jnp.sum(jnp.where(xbits > mid, 1, 0))
        small = cnt < _PRE_N
        return (jnp.where(small, lo, mid), jnp.where(small, mid, hi))

    lo0 = jnp.int32(-1)
    hi0 = jnp.int32(0x3F800000)
    _, hi = jax.lax.fori_loop(0, 31, body, (lo0, hi0))
    n_gt = jnp.sum(jnp.where(xbits > hi, 1, 0))
    n_ge = jnp.sum(jnp.where(xbits >= hi, 1, 0))
    l = jax.lax.broadcasted_iota(jnp.int32, (1, 128), 1)
    out_ref[0] = jnp.where(l == 0, hi, jnp.where(l == 1, n_gt, n_ge))

    dn = (((1,), (0,)), ((), ()))
    r128 = jax.lax.broadcasted_iota(jnp.int32, (128, 128), 0)
    c128 = jax.lax.broadcasted_iota(jnp.int32, (128, 128), 1)
    ul = jnp.where(r128 <= c128, 1.0, 0.0)
    r288 = jax.lax.broadcasted_iota(jnp.int32, (288, 288), 0)
    c288 = jax.lax.broadcasted_iota(jnp.int32, (288, 288), 1)
    tl = jnp.where(r288 > c288, 1.0, 0.0)
    eqf = jnp.where(xbits == hi, 1.0, 0.0)
    pe = jax.lax.dot_general(eqf, ul, dn, preferred_element_type=jnp.float32)
    oe = jax.lax.dot_general(tl, pe[:, 127:128], dn,
                             preferred_element_type=jnp.float32)
    eqrank = pe + oe
    kneed = jnp.float32(_PRE_N) - n_gt.astype(jnp.float32)
    win = jnp.where((xbits > hi) | ((xbits == hi) & (eqrank <= kneed)),
                    1.0, 0.0)
    pw = jax.lax.dot_general(win, ul, dn, preferred_element_type=jnp.float32)
    ow = jax.lax.dot_general(tl, pw[:, 127:128], dn,
                             preferred_element_type=jnp.float32)
    cwin_ref[0] = (pw + ow).astype(jnp.int32)


def _thr_pallas(xbits):
    return pl.pallas_call(
        _thr_body,
        grid=(_B,),
        in_specs=[pl.BlockSpec((1, 288, 128), lambda i: (i, 0, 0))],
        out_specs=[
            pl.BlockSpec((1, 1, 128), lambda i: (i, 0, 0)),
            pl.BlockSpec((1, 288, 128), lambda i: (i, 0, 0)),
        ],
        out_shape=[
            jax.ShapeDtypeStruct((_B, 1, 128), jnp.int32),
            jax.ShapeDtypeStruct((_B, 288, 128), jnp.int32),
        ],
        interpret=_INTERPRET,
    )(xbits)


def _topk2000(obj, proposals):
    nf = jnp.float32(-jnp.inf)
    xbits = jax.lax.bitcast_convert_type(obj, jnp.int32).reshape(_B, 288, 128)
    out, cwin3 = _thr_pallas(xbits)
    n_ge = out[:, 0, 2]
    cwin = cwin3.reshape(_B, _HW * _A)

    def fast(obj):
        p = jnp.broadcast_to(jnp.arange(1, _NPAD + 1, dtype=jnp.int32)[None],
                             (_B, _NPAD))
        pos = jax.vmap(lambda c, q: jnp.searchsorted(c, q))(cwin, p)
        posc = jnp.minimum(pos, _HW * _A - 1).astype(jnp.int32)
        wsc = jnp.where(p <= _PRE_N,
                        jnp.take_along_axis(obj, posc, axis=1), nf)
        sc, order = jax.lax.top_k(wsc, _PRE_N)
        idx = jnp.take_along_axis(posc, order, axis=1)
        return sc, idx

    def slow(obj):
        v, i = jax.lax.top_k(obj, _PRE_N)
        return v, i

    sc, idx = jax.lax.cond(jnp.any(n_ge > _NPAD), slow, fast, obj)
    bsel = jnp.take_along_axis(proposals, idx[..., None], axis=1)
    return sc, bsel


def _conv_xla(x, w, b):
    out = jax.lax.conv_general_dilated(
        x, w, (1, 1), 'SAME', dimension_numbers=('NCHW', 'OIHW', 'NCHW'))
    return out + b[None, :, None, None]


def _mk_anchors(image_size, fh, fw):
    sizes = jnp.array([32.0, 64.0, 128.0], dtype=jnp.float32)
    ratios = jnp.array([0.5, 1.0, 2.0], dtype=jnp.float32)
    ws = (sizes[:, None] * jnp.sqrt(ratios)[None, :]).reshape(-1)
    hs = (sizes[:, None] / jnp.sqrt(ratios)[None, :]).reshape(-1)
    sy = image_size / fh
    sx = image_size / fw
    cy = (jnp.arange(fh, dtype=jnp.float32) + 0.5) * sy
    cx = (jnp.arange(fw, dtype=jnp.float32) + 0.5) * sx
    cyg, cxg = jnp.meshgrid(cy, cx, indexing='ij')
    x1 = cxg[:, :, None] - ws[None, None, :] * 0.5
    y1 = cyg[:, :, None] - hs[None, None, :] * 0.5
    x2 = cxg[:, :, None] + ws[None, None, :] * 0.5
    y2 = cyg[:, :, None] + hs[None, None, :] * 0.5
    return jnp.stack([x1, y1, x2, y2], axis=-1).reshape(-1, 4)


def _decode(anchors, deltas):
    wa = anchors[:, 2] - anchors[:, 0]
    ha = anchors[:, 3] - anchors[:, 1]
    cxa = anchors[:, 0] + 0.5 * wa
    cya = anchors[:, 1] + 0.5 * ha
    dx, dy = deltas[:, 0], deltas[:, 1]
    dw = jnp.minimum(deltas[:, 2], 4.135)
    dh = jnp.minimum(deltas[:, 3], 4.135)
    cx = dx * wa + cxa
    cy = dy * ha + cya
    w = jnp.exp(dw) * wa
    h = jnp.exp(dh) * ha
    return jnp.stack(
        [cx - 0.5 * w, cy - 0.5 * h, cx + 0.5 * w, cy + 0.5 * h], axis=1)


def kernel(features, w1, b1, w_cls, b_cls, w_reg, b_reg, image_size):
    bsz = features.shape[0]
    image_size_f = jnp.asarray(image_size, dtype=jnp.float32)

    x = jnp.transpose(features, (0, 2, 3, 1)).reshape(bsz, _HW, _C)
    xp = jnp.pad(x, ((0, 0), (_PADR, _PADR), (0, 0)))
    w1t = jnp.transpose(w1, (2, 3, 1, 0)).reshape(9 * _C, _C)
    wh = jnp.concatenate([w_reg[:, :, 0, 0].T, w_cls[:, :, 0, 0].T], axis=1)
    bh = jnp.concatenate([b_reg, b_cls])[None, :]

    anch = _mk_anchors(image_size_f, _FH, _FW)
    wa = (anch[:, 2] - anch[:, 0]).reshape(_HW, _A)
    ha = (anch[:, 3] - anch[:, 1]).reshape(_HW, _A)
    cxa = anch[:, 0].reshape(_HW, _A) + 0.5 * wa
    cya = anch[:, 1].reshape(_HW, _A) + 0.5 * ha
    sa = jnp.stack([wa, ha, wa, ha], axis=-1).reshape(1, _HW, 36)
    ca = jnp.stack([cxa, cya, cxa, cya], axis=-1).reshape(1, _HW, 36)
    imsz = image_size_f.reshape(1, 1, 1)

    obj9, prop36 = _rpn_front(xp, w1t, wh, b1[None, :], bh, ca, sa, imsz)
    obj = obj9.reshape(bsz, _HW * _A)
    proposals = prop36.reshape(bsz, _HW * _A, 4)

    sc, bsel = _topk2000(obj, proposals)
    pad = _NPAD - _PRE_N
    bpad = jnp.pad(bsel, ((0, 0), (0, pad), (0, 0)))
    scpad = jnp.pad(sc, ((0, 0), (0, pad)), constant_values=0.0)
    bt = jnp.transpose(bpad, (0, 2, 1))
    x1c = bpad[:, :, 0:1]
    y1c = bpad[:, :, 1:2]
    x2c = bpad[:, :, 2:3]
    y2c = bpad[:, :, 3:4]
    masked = _nms_pallas(bt, x1c, y1c, x2c, y2c, scpad[:, None, :])
    masked = masked[:, 0, :_PRE_N]
    _, kidx = jax.lax.top_k(masked, _POST_N)
    return jnp.take_along_axis(bsel, kidx[..., None], axis=1)


# speedup vs baseline: 1.1830x; 1.0011x over previous
import jax
import jax.numpy as jnp
from jax.experimental import pallas as pl
from jax.experimental.pallas import tpu as pltpu

_B, _C, _FH, _FW = 2, 256, 64, 64
_A = 9
_PRE_N, _POST_N = 2000, 1000
_NPAD = 2048
_BLK = 128
_NBLK = _NPAD // _BLK

def _nms_body(bt_ref, x1c_ref, y1c_ref, x2c_ref, y2c_ref, sc_ref, out_ref):
    x1r = bt_ref[0, 0:1, :]
    y1r = bt_ref[0, 1:2, :]
    x2r = bt_ref[0, 2:3, :]
    y2r = bt_ref[0, 3:4, :]
    arear = (x2r - x1r) * (y2r - y1r)
    gcol = jax.lax.broadcasted_iota(jnp.int32, (1, _NPAD), 1)
    rio = jax.lax.broadcasted_iota(jnp.int32, (_BLK, _BLK), 0)
    cio = jax.lax.broadcasted_iota(jnp.int32, (_BLK, _BLK), 1)
    tri = cio > rio
    dead = jnp.zeros((1, _NPAD), jnp.float32)
    keeps = []
    for b in range(_NBLK):
        base = b * _BLK
        x1c = x1c_ref[0, pl.ds(base, _BLK), :]
        y1c = y1c_ref[0, pl.ds(base, _BLK), :]
        x2c = x2c_ref[0, pl.ds(base, _BLK), :]
        y2c = y2c_ref[0, pl.ds(base, _BLK), :]
        iw = jnp.maximum(jnp.minimum(x2c, x2r) - jnp.maximum(x1c, x1r), 0.0)
        ih = jnp.maximum(jnp.minimum(y2c, y2r) - jnp.maximum(y1c, y1r), 0.0)
        inter = iw * ih
        areac = (x2c - x1c) * (y2c - y1c)
        thr = 0.7 * (areac + arear + 1e-8)
        swide = jnp.where(1.7 * inter > thr, 1.0, 0.0)
        supblk = jnp.where(tri, swide[:, base:base + _BLK], 0.0)
        keep0 = 1.0 - dead[0:1, base:base + _BLK]

        def w_cond(c):
            return c[1]

        def w_body(c):
            x = c[0]
            xb = jnp.broadcast_to(x, (8, _BLK))
            cnt = jax.lax.dot_general(
                xb, supblk, (((1,), (0,)), ((), ())),
                preferred_element_type=jnp.float32)[0:1, :]
            xn = jnp.where((cnt < 0.5) & (keep0 > 0.5), 1.0, 0.0)
            return (xn, jnp.any(xn != x))

        keep, _ = jax.lax.while_loop(w_cond, w_body,
                                     (keep0, jnp.bool_(True)))
        keeps.append(keep)
        if b < _NBLK - 1:
            km = jnp.broadcast_to(keep, (8, _BLK))
            cnt = jax.lax.dot_general(km, swide, (((1,), (0,)), ((), ())),
                                      preferred_element_type=jnp.float32)
            live = (cnt[0:1, :] > 0.5) & (gcol >= base + _BLK)
            dead = jnp.maximum(dead, jnp.where(live, 1.0, 0.0))
    keep_full = jnp.concatenate(keeps, axis=1)
    sc = sc_ref[0]
    out_ref[0] = jnp.where(keep_full > 0.5, sc, -jnp.inf)


def _nms_pallas(bt, x1c, y1c, x2c, y2c, sc):
    spec3 = lambda shape: pl.BlockSpec(shape, lambda i: (i, 0, 0))
    return pl.pallas_call(
        _nms_body,
        grid=(_B,),
        in_specs=[
            spec3((1, 4, _NPAD)),
            spec3((1, _NPAD, 1)),
            spec3((1, _NPAD, 1)),
            spec3((1, _NPAD, 1)),
            spec3((1, _NPAD, 1)),
            spec3((1, 1, _NPAD)),
        ],
        out_specs=spec3((1, 1, _NPAD)),
        out_shape=jax.ShapeDtypeStruct((_B, 1, _NPAD), jnp.float32),
    )(bt, x1c, y1c, x2c, y2c, sc)


_HW = _FH * _FW
_RH = 512
_NSTRIP = _HW // _RH
_PADR = 65


def _rpn_front_body(xp_ref, w1_ref, wh_ref, b1_ref, bh_ref, ca_ref, sa_ref,
                    imsz_ref, obj_ref, prop_ref):
    b1r = b1_ref[0:1, :]
    bhr = bh_ref[0:1, :]
    imv = imsz_ref[0, :, :]
    rio = jax.lax.broadcasted_iota(jnp.int32, (_RH, 1), 0)
    w_of_r = jax.lax.rem(rio, _FW)
    m0 = jnp.where(w_of_r == 0, 0.0, 1.0)
    m2 = jnp.where(w_of_r == _FW - 1, 0.0, 1.0)
    jl = jax.lax.broadcasted_iota(jnp.int32, (1, 36), 1) % 4
    sgn = jnp.where(jl < 2, -0.5, 0.5)
    for i in range(_NSTRIP):
        acc = jnp.zeros((_RH, _C), jnp.float32)
        for dy in range(3):
            for dx in range(3):
                start = i * _RH + dy * _FW + dx
                xs = xp_ref[0, start:start + _RH, :]
                if dx == 0:
                    xs = xs * m0
                elif dx == 2:
                    xs = xs * m2
                wt = w1_ref[(dy * 3 + dx) * _C:(dy * 3 + dx + 1) * _C, :]
                acc = acc + jax.lax.dot_general(
                    xs, wt, (((1,), (0,)), ((), ())),
                    preferred_element_type=jnp.float32)
        t = jnp.maximum(acc + b1r, 0.0)
        head = jax.lax.dot_general(t, wh_ref[:, :],
                                   (((1,), (0,)), ((), ())),
                                   preferred_element_type=jnp.float32) + bhr
        logits = head[:, 36:45]
        z = jnp.exp(-jnp.abs(logits))
        sig = jnp.where(logits >= 0, 1.0 / (1.0 + z), z / (1.0 + z))
        obj_ref[0, i * _RH:(i + 1) * _RH, :] = sig
        a = head[:, 0:36]
        a2 = jnp.concatenate([a[:, 2:], a[:, :2]], axis=1)
        am2 = jnp.concatenate([a[:, 34:], a[:, :34]], axis=1)
        b2 = jnp.where(jl < 2, a2, a)
        b0 = jnp.where(jl < 2, a, am2)
        ca = ca_ref[0, i * _RH:(i + 1) * _RH, :]
        sa = sa_ref[0, i * _RH:(i + 1) * _RH, :]
        e = jnp.exp(jnp.minimum(b2, 4.135)) * sa
        prop = b0 * sa + ca + sgn * e
        prop_ref[0, i * _RH:(i + 1) * _RH, :] = jnp.minimum(
            jnp.maximum(prop, 0.0), imv)


def _rpn_front(xp, w1t, wh, b1, bh, ca, sa, imsz):
    bspec = lambda shape: pl.BlockSpec(shape, lambda i: (i,) + (0,) * (len(shape) - 1))
    cspec = lambda shape: pl.BlockSpec(shape, lambda i: (0,) * len(shape))
    return pl.pallas_call(
        _rpn_front_body,
        grid=(_B,),
        in_specs=[
            bspec((1, _HW + 2 * _PADR, _C)),
            cspec((9 * _C, _C)),
            cspec((_C, 45)),
            cspec((1, _C)),
            cspec((1, 45)),
            cspec((1, _HW, 36)),
            cspec((1, _HW, 36)),
            cspec((1, 1, 1)),
        ],
        out_specs=[bspec((1, _HW, 9)), bspec((1, _HW, 36))],
        out_shape=[
            jax.ShapeDtypeStruct((_B, _HW, 9), jnp.float32),
            jax.ShapeDtypeStruct((_B, _HW, 36), jnp.float32),
        ],
    )(xp, w1t, wh, b1, bh, ca, sa, imsz)


def _thr_body(xb_ref, out_ref, cwin_ref):
    xbits = xb_ref[0]

    def body(_, c):
        lo, hi = c
        mid = jax.lax.div(lo + hi, jnp.int32(2))
        cnt = jnp.sum(jnp.where(xbits > mid, 1, 0))
        small = cnt < _PRE_N
        return (jnp.where(small, lo, mid), jnp.where(small, mid, hi))

    lo0 = jnp.int32(-1)
    hi0 = jnp.int32(0x3F800000)
    _, hi = jax.lax.fori_loop(0, 31, body, (lo0, hi0))
    n_gt = jnp.sum(jnp.where(xbits > hi, 1, 0))
    n_ge = jnp.sum(jnp.where(xbits >= hi, 1, 0))
    l = jax.lax.broadcasted_iota(jnp.int32, (1, 128), 1)
    out_ref[0] = jnp.where(l == 0, hi, jnp.where(l == 1, n_gt, n_ge))

    dn = (((1,), (0,)), ((), ()))
    r128 = jax.lax.broadcasted_iota(jnp.int32, (128, 128), 0)
    c128 = jax.lax.broadcasted_iota(jnp.int32, (128, 128), 1)
    ul = jnp.where(r128 <= c128, 1.0, 0.0)
    r288 = jax.lax.broadcasted_iota(jnp.int32, (288, 288), 0)
    c288 = jax.lax.broadcasted_iota(jnp.int32, (288, 288), 1)
    tl = jnp.where(r288 > c288, 1.0, 0.0)
    eqf = jnp.where(xbits == hi, 1.0, 0.0)
    pe = jax.lax.dot_general(eqf, ul, dn, preferred_element_type=jnp.float32)
    oe = jax.lax.dot_general(tl, pe[:, 127:128], dn,
                             preferred_element_type=jnp.float32)
    eqrank = pe + oe
    kneed = jnp.float32(_PRE_N) - n_gt.astype(jnp.float32)
    win = jnp.where((xbits > hi) | ((xbits == hi) & (eqrank <= kneed)),
                    1.0, 0.0)
    pw = jax.lax.dot_general(win, ul, dn, preferred_element_type=jnp.float32)
    ow = jax.lax.dot_general(tl, pw[:, 127:128], dn,
                             preferred_element_type=jnp.float32)
    cwin_ref[0] = (pw + ow).astype(jnp.int32)


def _thr_pallas(xbits):
    return pl.pallas_call(
        _thr_body,
        grid=(_B,),
        in_specs=[pl.BlockSpec((1, 288, 128), lambda i: (i, 0, 0))],
        out_specs=[
            pl.BlockSpec((1, 1, 128), lambda i: (i, 0, 0)),
            pl.BlockSpec((1, 288, 128), lambda i: (i, 0, 0)),
        ],
        out_shape=[
            jax.ShapeDtypeStruct((_B, 1, 128), jnp.int32),
            jax.ShapeDtypeStruct((_B, 288, 128), jnp.int32),
        ],
    )(xbits)


def _topk2000(obj, proposals):
    nf = jnp.float32(-jnp.inf)
    xbits = jax.lax.bitcast_convert_type(obj, jnp.int32).reshape(_B, 288, 128)
    out, cwin3 = _thr_pallas(xbits)
    n_ge = out[:, 0, 2]
    cwin = cwin3.reshape(_B, _HW * _A)

    def fast(obj):
        p = jnp.broadcast_to(jnp.arange(1, _NPAD + 1, dtype=jnp.int32)[None],
                             (_B, _NPAD))
        pos = jax.vmap(lambda c, q: jnp.searchsorted(c, q))(cwin, p)
        posc = jnp.minimum(pos, _HW * _A - 1).astype(jnp.int32)
        wsc = jnp.where(p <= _PRE_N,
                        jnp.take_along_axis(obj, posc, axis=1), nf)
        sc, order = jax.lax.top_k(wsc, _PRE_N)
        idx = jnp.take_along_axis(posc, order, axis=1)
        return sc, idx

    def slow(obj):
        v, i = jax.lax.top_k(obj, _PRE_N)
        return v, i

    sc, idx = jax.lax.cond(jnp.any(n_ge > _NPAD), slow, fast, obj)
    bsel = jnp.take_along_axis(proposals, idx[..., None], axis=1)
    return sc, bsel


def _mk_anchors(image_size, fh, fw):
    sizes = jnp.array([32.0, 64.0, 128.0], dtype=jnp.float32)
    ratios = jnp.array([0.5, 1.0, 2.0], dtype=jnp.float32)
    ws = (sizes[:, None] * jnp.sqrt(ratios)[None, :]).reshape(-1)
    hs = (sizes[:, None] / jnp.sqrt(ratios)[None, :]).reshape(-1)
    sy = image_size / fh
    sx = image_size / fw
    cy = (jnp.arange(fh, dtype=jnp.float32) + 0.5) * sy
    cx = (jnp.arange(fw, dtype=jnp.float32) + 0.5) * sx
    cyg, cxg = jnp.meshgrid(cy, cx, indexing='ij')
    x1 = cxg[:, :, None] - ws[None, None, :] * 0.5
    y1 = cyg[:, :, None] - hs[None, None, :] * 0.5
    x2 = cxg[:, :, None] + ws[None, None, :] * 0.5
    y2 = cyg[:, :, None] + hs[None, None, :] * 0.5
    return jnp.stack([x1, y1, x2, y2], axis=-1).reshape(-1, 4)


def kernel(features, w1, b1, w_cls, b_cls, w_reg, b_reg, image_size):
    bsz = features.shape[0]
    image_size_f = jnp.asarray(image_size, dtype=jnp.float32)

    x = jnp.transpose(features, (0, 2, 3, 1)).reshape(bsz, _HW, _C)
    xp = jnp.pad(x, ((0, 0), (_PADR, _PADR), (0, 0)))
    w1t = jnp.transpose(w1, (2, 3, 1, 0)).reshape(9 * _C, _C)
    wh = jnp.concatenate([w_reg[:, :, 0, 0].T, w_cls[:, :, 0, 0].T], axis=1)
    bh = jnp.concatenate([b_reg, b_cls])[None, :]

    anch = _mk_anchors(image_size_f, _FH, _FW)
    wa = (anch[:, 2] - anch[:, 0]).reshape(_HW, _A)
    ha = (anch[:, 3] - anch[:, 1]).reshape(_HW, _A)
    cxa = anch[:, 0].reshape(_HW, _A) + 0.5 * wa
    cya = anch[:, 1].reshape(_HW, _A) + 0.5 * ha
    sa = jnp.stack([wa, ha, wa, ha], axis=-1).reshape(1, _HW, 36)
    ca = jnp.stack([cxa, cya, cxa, cya], axis=-1).reshape(1, _HW, 36)
    imsz = image_size_f.reshape(1, 1, 1)

    obj9, prop36 = _rpn_front(xp, w1t, wh, b1[None, :], bh, ca, sa, imsz)
    obj = obj9.reshape(bsz, _HW * _A)
    proposals = prop36.reshape(bsz, _HW * _A, 4)

    sc, bsel = _topk2000(obj, proposals)
    pad = _NPAD - _PRE_N
    bpad = jnp.pad(bsel, ((0, 0), (0, pad), (0, 0)))
    scpad = jnp.pad(sc, ((0, 0), (0, pad)), constant_values=0.0)
    bt = jnp.transpose(bpad, (0, 2, 1))
    x1c = bpad[:, :, 0:1]
    y1c = bpad[:, :, 1:2]
    x2c = bpad[:, :, 2:3]
    y2c = bpad[:, :, 3:4]
    masked = _nms_pallas(bt, x1c, y1c, x2c, y2c, scpad[:, None, :])
    masked = masked[:, 0, :_PRE_N]
    _, kidx = jax.lax.top_k(masked, _POST_N)
    return jnp.take_along_axis(bsel, kidx[..., None], axis=1)
